# Initial kernel scaffold; baseline (speedup 1.0000x reference)
import functools

import jax
import jax.numpy as jnp
from jax import lax
from jax.experimental import pallas as pl
from jax.experimental.pallas import tpu as pltpu
from jax.experimental.pallas import tpu_sc as plsc

D_MODEL = 2048
D_FF = 4096
E = 16
SEQ = 2048
CAP = 320
NSLOT = E * CAP
TRASH = NSLOT
NSLOT_PAD = NSLOT + 8

NC = 2
NS = 16
NW = NC * NS
TOK_PER_W = SEQ // NW
CHUNK = 16


def _routing_body(x_ref, wg_ref, slot_ref, w1_ref, w2_ref, laux_ref, cnt_ref):
    x = x_ref[...]
    wg = wg_ref[...]
    logits = jnp.dot(x, wg, preferred_element_type=jnp.float32)

    m = jnp.max(logits, axis=1, keepdims=True)
    eg = jnp.exp(logits - m)
    gates = eg / jnp.sum(eg, axis=1, keepdims=True)

    lane = lax.broadcasted_iota(jnp.int32, (SEQ, E), 1)
    idx1 = jnp.min(jnp.where(logits == m, lane, E), axis=1, keepdims=True)
    mask1 = (lane == idx1).astype(jnp.float32)
    logits2 = jnp.where(mask1 > 0, -jnp.inf, logits)
    m2 = jnp.max(logits2, axis=1, keepdims=True)
    idx2 = jnp.min(jnp.where(logits2 == m2, lane, E), axis=1, keepdims=True)
    mask2 = (lane == idx2).astype(jnp.float32)

    row = lax.broadcasted_iota(jnp.int32, (SEQ, SEQ), 0)
    col = lax.broadcasted_iota(jnp.int32, (SEQ, SEQ), 1)
    tri = (col <= row).astype(jnp.float32)
    cs1 = jnp.dot(tri, mask1, preferred_element_type=jnp.float32)
    cs2 = jnp.dot(tri, mask2, preferred_element_type=jnp.float32)
    n1 = jnp.sum(mask1, axis=0, keepdims=True)
    loc1 = cs1 - 1.0
    loc2 = cs2 - 1.0 + n1

    me = jnp.mean(gates, axis=0)
    ce = jnp.mean(mask1, axis=0)
    laux_ref[0, 0] = jnp.sum(me * ce) * float(E * E)

    mask1d = mask1 * (loc1 < CAP).astype(jnp.float32)
    mask2d = mask2 * (loc2 < CAP).astype(jnp.float32)
    pos1 = jnp.sum(loc1 * mask1d, axis=1, keepdims=True).astype(jnp.int32)
    pos2 = jnp.sum(loc2 * mask2d, axis=1, keepdims=True).astype(jnp.int32)
    keep1 = jnp.sum(mask1d, axis=1, keepdims=True)
    keep2 = jnp.sum(mask2d, axis=1, keepdims=True)

    g1 = jnp.sum(gates * mask1d, axis=1, keepdims=True)
    g2 = jnp.sum(gates * mask2d, axis=1, keepdims=True)
    denom = g1 + g2
    denom = jnp.where(denom < 1e-9, 1.0, denom)
    w1_ref[...] = g1 / denom
    w2_ref[...] = g2 / denom

    cnt_ref[...] = jnp.sum(mask1d + mask2d, axis=0, keepdims=True).astype(jnp.int32)

    s1 = jnp.where(keep1 > 0, idx1 * CAP + pos1, TRASH)
    s2 = jnp.where(keep2 > 0, idx2 * CAP + pos2, TRASH)
    slot_ref[:, 0:1] = s1
    slot_ref[:, 1:2] = s2


def _routing(x, wg):
    return pl.pallas_call(
        _routing_body,
        out_shape=(
            jax.ShapeDtypeStruct((SEQ, 2), jnp.int32),
            jax.ShapeDtypeStruct((SEQ, 1), jnp.float32),
            jax.ShapeDtypeStruct((SEQ, 1), jnp.float32),
            jax.ShapeDtypeStruct((1, 1), jnp.float32),
            jax.ShapeDtypeStruct((1, E), jnp.int32),
        ),
    )(x, wg)


def _dispatch_body(x_hbm, slot_hbm, eout_hbm, buf, idx1_v, idx2_v, sem):
    wid = lax.axis_index("s") * NC + lax.axis_index("c")
    base = wid * TOK_PER_W
    for ch in range(TOK_PER_W // CHUNK):
        off = base + ch * CHUNK
        pltpu.sync_copy(x_hbm.at[pl.ds(off, CHUNK)], buf)
        pltpu.sync_copy(slot_hbm.at[pl.ds(off, CHUNK), 0], idx1_v)
        pltpu.sync_copy(slot_hbm.at[pl.ds(off, CHUNK), 1], idx2_v)
        c1 = pltpu.async_copy(buf, eout_hbm.at[idx1_v], sem)
        c2 = pltpu.async_copy(buf, eout_hbm.at[idx2_v], sem)
        c1.wait()
        c2.wait()


def _dispatch(x, slots):
    mesh = plsc.VectorSubcoreMesh(core_axis_name="c", subcore_axis_name="s")
    return pl.kernel(
        _dispatch_body,
        out_type=jax.ShapeDtypeStruct((NSLOT_PAD, D_MODEL), jnp.float32),
        mesh=mesh,
        scratch_types=[
            pltpu.VMEM((CHUNK, D_MODEL), jnp.float32),
            pltpu.VMEM((CHUNK,), jnp.int32),
            pltpu.VMEM((CHUNK,), jnp.int32),
            pltpu.SemaphoreType.DMA,
        ],
    )(x, slots)


F_BLK = 1024
NF = D_FF // F_BLK


def _ffn_body(a_ref, w1_ref, b1_ref, w2_ref, b2_ref, out_ref):
    f = pl.program_id(1)
    h = jnp.dot(a_ref[...], w1_ref[0], preferred_element_type=jnp.float32)
    h = jnp.maximum(h + b1_ref[...], 0.0)
    contrib = jnp.dot(h, w2_ref[0], preferred_element_type=jnp.float32)

    @pl.when(f == 0)
    def _():
        out_ref[...] = contrib + b2_ref[...]

    @pl.when(f != 0)
    def _():
        out_ref[...] += contrib


def _ffn(expert_in, w1, b1, w2, b2):
    return pl.pallas_call(
        _ffn_body,
        grid=(E, NF),
        in_specs=[
            pl.BlockSpec((CAP, D_MODEL), lambda e, f: (e, 0)),
            pl.BlockSpec((1, D_MODEL, F_BLK), lambda e, f: (e, 0, f)),
            pl.BlockSpec((1, F_BLK), lambda e, f: (e, f)),
            pl.BlockSpec((1, F_BLK, D_MODEL), lambda e, f: (e, f, 0)),
            pl.BlockSpec((1, D_MODEL), lambda e, f: (e, 0)),
        ],
        out_specs=pl.BlockSpec((CAP, D_MODEL), lambda e, f: (e, 0)),
        out_shape=jax.ShapeDtypeStruct((NSLOT_PAD, D_MODEL), jnp.float32),
        compiler_params=pltpu.CompilerParams(
            dimension_semantics=("arbitrary", "arbitrary"),
        ),
    )(expert_in, w1, b1, w2, b2)


def _gather_body(eo_hbm, slot_hbm, g1_hbm, g2_hbm, buf1, buf2, idx1_v, idx2_v, sem):
    wid = lax.axis_index("s") * NC + lax.axis_index("c")
    base = wid * TOK_PER_W
    for ch in range(TOK_PER_W // CHUNK):
        off = base + ch * CHUNK
        pltpu.sync_copy(slot_hbm.at[pl.ds(off, CHUNK), 0], idx1_v)
        pltpu.sync_copy(slot_hbm.at[pl.ds(off, CHUNK), 1], idx2_v)
        c1 = pltpu.async_copy(eo_hbm.at[idx1_v], buf1, sem)
        c2 = pltpu.async_copy(eo_hbm.at[idx2_v], buf2, sem)
        c1.wait()
        c2.wait()
        pltpu.sync_copy(buf1, g1_hbm.at[pl.ds(off, CHUNK)])
        pltpu.sync_copy(buf2, g2_hbm.at[pl.ds(off, CHUNK)])


def _gather(expert_out, slots):
    mesh = plsc.VectorSubcoreMesh(core_axis_name="c", subcore_axis_name="s")
    return pl.kernel(
        _gather_body,
        out_type=(
            jax.ShapeDtypeStruct((SEQ, D_MODEL), jnp.float32),
            jax.ShapeDtypeStruct((SEQ, D_MODEL), jnp.float32),
        ),
        mesh=mesh,
        scratch_types=[
            pltpu.VMEM((CHUNK, D_MODEL), jnp.float32),
            pltpu.VMEM((CHUNK, D_MODEL), jnp.float32),
            pltpu.VMEM((CHUNK,), jnp.int32),
            pltpu.VMEM((CHUNK,), jnp.int32),
            pltpu.SemaphoreType.DMA,
        ],
    )(expert_out, slots)


ROW_BLK = 256


def _combine_body(r1_ref, r2_ref, w1_ref, w2_ref, out_ref):
    w1 = w1_ref[...]
    w2 = w2_ref[...]
    t1 = jnp.where(w1 == 0.0, 0.0, w1 * r1_ref[...])
    t2 = jnp.where(w2 == 0.0, 0.0, w2 * r2_ref[...])
    out_ref[...] = t1 + t2


def _combine(rows1, rows2, w1, w2):
    return pl.pallas_call(
        _combine_body,
        grid=(SEQ // ROW_BLK,),
        in_specs=[
            pl.BlockSpec((ROW_BLK, D_MODEL), lambda i: (i, 0)),
            pl.BlockSpec((ROW_BLK, D_MODEL), lambda i: (i, 0)),
            pl.BlockSpec((ROW_BLK, 1), lambda i: (i, 0)),
            pl.BlockSpec((ROW_BLK, 1), lambda i: (i, 0)),
        ],
        out_specs=pl.BlockSpec((ROW_BLK, D_MODEL), lambda i: (i, 0)),
        out_shape=jax.ShapeDtypeStruct((SEQ, D_MODEL), jnp.float32),
    )(rows1, rows2, w1, w2)


def kernel(hidden_states, Wg, W1, b1, W2, b2):
    x = hidden_states.reshape(SEQ, D_MODEL)
    slots, w1, w2, laux, counts = _routing(x, Wg)
    expert_in = _dispatch(x, slots)
    expert_out = _ffn(expert_in, W1, b1, W2, b2)
    rows1, rows2 = _gather(expert_out, slots)
    out = _combine(rows1, rows2, w1, w2)
    return out.reshape(hidden_states.shape), laux[0, 0], counts.reshape(E)


# speedup vs baseline: 1.2766x; 1.2766x over previous
import functools

import jax
import jax.numpy as jnp
from jax import lax
from jax.experimental import pallas as pl
from jax.experimental.pallas import tpu as pltpu
from jax.experimental.pallas import tpu_sc as plsc

D_MODEL = 2048
D_FF = 4096
E = 16
SEQ = 2048
CAP = 320
NSLOT = E * CAP
TRASH = NSLOT
NSLOT_PAD = NSLOT + 8

NC = 2
NS = 16
NW = NC * NS
TOK_PER_W = SEQ // NW
CHUNK = 16


def _routing_body(x_ref, wg_ref, slot1_ref, slot2_ref, w1_ref, w2_ref, laux_ref,
                  cnt_ref):
    x = x_ref[...]
    wg = wg_ref[...]
    logits = jnp.dot(x, wg, preferred_element_type=jnp.float32)

    m = jnp.max(logits, axis=1, keepdims=True)
    eg = jnp.exp(logits - m)
    gates = eg / jnp.sum(eg, axis=1, keepdims=True)

    lane = lax.broadcasted_iota(jnp.int32, (SEQ, E), 1)
    idx1 = jnp.min(jnp.where(logits == m, lane, E), axis=1, keepdims=True)
    mask1 = (lane == idx1).astype(jnp.float32)
    logits2 = jnp.where(mask1 > 0, -jnp.inf, logits)
    m2 = jnp.max(logits2, axis=1, keepdims=True)
    idx2 = jnp.min(jnp.where(logits2 == m2, lane, E), axis=1, keepdims=True)
    mask2 = (lane == idx2).astype(jnp.float32)

    row = lax.broadcasted_iota(jnp.int32, (SEQ, SEQ), 0)
    col = lax.broadcasted_iota(jnp.int32, (SEQ, SEQ), 1)
    tri = (col <= row).astype(jnp.float32)
    cs1 = jnp.dot(tri, mask1, preferred_element_type=jnp.float32)
    cs2 = jnp.dot(tri, mask2, preferred_element_type=jnp.float32)
    n1 = jnp.sum(mask1, axis=0, keepdims=True)
    loc1 = cs1 - 1.0
    loc2 = cs2 - 1.0 + n1

    me = jnp.mean(gates, axis=0, keepdims=True)
    ce = jnp.mean(mask1, axis=0, keepdims=True)
    laux_ref[...] = jnp.sum(me * ce, axis=1, keepdims=True) * float(E * E)

    mask1d = mask1 * (loc1 < CAP).astype(jnp.float32)
    mask2d = mask2 * (loc2 < CAP).astype(jnp.float32)
    pos1 = jnp.sum(loc1 * mask1d, axis=1, keepdims=True).astype(jnp.int32)
    pos2 = jnp.sum(loc2 * mask2d, axis=1, keepdims=True).astype(jnp.int32)
    keep1 = jnp.sum(mask1d, axis=1, keepdims=True)
    keep2 = jnp.sum(mask2d, axis=1, keepdims=True)

    g1 = jnp.sum(gates * mask1d, axis=1, keepdims=True)
    g2 = jnp.sum(gates * mask2d, axis=1, keepdims=True)
    denom = g1 + g2
    denom = jnp.where(denom < 1e-9, 1.0, denom)
    w1_ref[...] = g1 / denom
    w2_ref[...] = g2 / denom

    cnt_ref[...] = jnp.sum(mask1d + mask2d, axis=0, keepdims=True).astype(jnp.int32)

    slot1_ref[...] = jnp.where(keep1 > 0, idx1 * CAP + pos1, TRASH)
    slot2_ref[...] = jnp.where(keep2 > 0, idx2 * CAP + pos2, TRASH)


def _routing(x, wg):
    return pl.pallas_call(
        _routing_body,
        out_shape=(
            jax.ShapeDtypeStruct((SEQ, 1), jnp.int32),
            jax.ShapeDtypeStruct((SEQ, 1), jnp.int32),
            jax.ShapeDtypeStruct((SEQ, 1), jnp.float32),
            jax.ShapeDtypeStruct((SEQ, 1), jnp.float32),
            jax.ShapeDtypeStruct((1, 1), jnp.float32),
            jax.ShapeDtypeStruct((1, E), jnp.int32),
        ),
    )(x, wg)


def _dispatch_body(x_hbm, slot1_hbm, slot2_hbm, eout_hbm, buf, idx1_v, idx2_v, sem):
    wid = lax.axis_index("s") * NC + lax.axis_index("c")
    base = wid * TOK_PER_W
    for ch in range(TOK_PER_W // CHUNK):
        off = base + ch * CHUNK
        pltpu.sync_copy(x_hbm.at[pl.ds(off, CHUNK)], buf)
        pltpu.sync_copy(slot1_hbm.at[pl.ds(off, CHUNK)], idx1_v)
        pltpu.sync_copy(slot2_hbm.at[pl.ds(off, CHUNK)], idx2_v)
        c1 = pltpu.async_copy(buf, eout_hbm.at[idx1_v], sem)
        c2 = pltpu.async_copy(buf, eout_hbm.at[idx2_v], sem)
        c1.wait()
        c2.wait()


def _dispatch(x, slot1, slot2):
    mesh = plsc.VectorSubcoreMesh(core_axis_name="c", subcore_axis_name="s")
    return pl.kernel(
        _dispatch_body,
        out_type=jax.ShapeDtypeStruct((NSLOT_PAD, D_MODEL), jnp.float32),
        mesh=mesh,
        scratch_types=[
            pltpu.VMEM((CHUNK, D_MODEL), jnp.float32),
            pltpu.VMEM((CHUNK,), jnp.int32),
            pltpu.VMEM((CHUNK,), jnp.int32),
            pltpu.SemaphoreType.DMA,
        ],
    )(x, slot1, slot2)


F_BLK = 1024
NF = D_FF // F_BLK


def _ffn_body(a_ref, w1_ref, b1_ref, w2_ref, b2_ref, out_ref):
    f = pl.program_id(1)
    h = jnp.dot(a_ref[...], w1_ref[0], preferred_element_type=jnp.float32)
    h = jnp.maximum(h + b1_ref[0], 0.0)
    contrib = jnp.dot(h, w2_ref[0], preferred_element_type=jnp.float32)

    @pl.when(f == 0)
    def _():
        out_ref[...] = contrib + b2_ref[0]

    @pl.when(f != 0)
    def _():
        out_ref[...] += contrib


def _ffn(expert_in, w1, b1, w2, b2):
    return pl.pallas_call(
        _ffn_body,
        grid=(E, NF),
        in_specs=[
            pl.BlockSpec((CAP, D_MODEL), lambda e, f: (e, 0)),
            pl.BlockSpec((1, D_MODEL, F_BLK), lambda e, f: (e, 0, f)),
            pl.BlockSpec((1, 1, F_BLK), lambda e, f: (e, 0, f)),
            pl.BlockSpec((1, F_BLK, D_MODEL), lambda e, f: (e, f, 0)),
            pl.BlockSpec((1, 1, D_MODEL), lambda e, f: (e, 0, 0)),
        ],
        out_specs=pl.BlockSpec((CAP, D_MODEL), lambda e, f: (e, 0)),
        out_shape=jax.ShapeDtypeStruct((NSLOT_PAD, D_MODEL), jnp.float32),
        compiler_params=pltpu.CompilerParams(
            dimension_semantics=("arbitrary", "arbitrary"),
        ),
    )(expert_in, w1, b1.reshape(E, 1, D_FF), w2, b2.reshape(E, 1, D_MODEL))


def _gather_body(eo_hbm, slot1_hbm, slot2_hbm, g1_hbm, g2_hbm, buf1, buf2,
                 idx1_v, idx2_v, sem):
    wid = lax.axis_index("s") * NC + lax.axis_index("c")
    base = wid * TOK_PER_W
    for ch in range(TOK_PER_W // CHUNK):
        off = base + ch * CHUNK
        pltpu.sync_copy(slot1_hbm.at[pl.ds(off, CHUNK)], idx1_v)
        pltpu.sync_copy(slot2_hbm.at[pl.ds(off, CHUNK)], idx2_v)
        c1 = pltpu.async_copy(eo_hbm.at[idx1_v], buf1, sem)
        c2 = pltpu.async_copy(eo_hbm.at[idx2_v], buf2, sem)
        c1.wait()
        c2.wait()
        pltpu.sync_copy(buf1, g1_hbm.at[pl.ds(off, CHUNK)])
        pltpu.sync_copy(buf2, g2_hbm.at[pl.ds(off, CHUNK)])


def _gather(expert_out, slot1, slot2):
    mesh = plsc.VectorSubcoreMesh(core_axis_name="c", subcore_axis_name="s")
    return pl.kernel(
        _gather_body,
        out_type=(
            jax.ShapeDtypeStruct((SEQ, D_MODEL), jnp.float32),
            jax.ShapeDtypeStruct((SEQ, D_MODEL), jnp.float32),
        ),
        mesh=mesh,
        scratch_types=[
            pltpu.VMEM((CHUNK, D_MODEL), jnp.float32),
            pltpu.VMEM((CHUNK, D_MODEL), jnp.float32),
            pltpu.VMEM((CHUNK,), jnp.int32),
            pltpu.VMEM((CHUNK,), jnp.int32),
            pltpu.SemaphoreType.DMA,
        ],
    )(expert_out, slot1, slot2)


ROW_BLK = 256


def _combine_body(r1_ref, r2_ref, w1_ref, w2_ref, out_ref):
    w1 = w1_ref[...]
    w2 = w2_ref[...]
    t1 = jnp.where(w1 == 0.0, 0.0, w1 * r1_ref[...])
    t2 = jnp.where(w2 == 0.0, 0.0, w2 * r2_ref[...])
    out_ref[...] = t1 + t2


def _combine(rows1, rows2, w1, w2):
    return pl.pallas_call(
        _combine_body,
        grid=(SEQ // ROW_BLK,),
        in_specs=[
            pl.BlockSpec((ROW_BLK, D_MODEL), lambda i: (i, 0)),
            pl.BlockSpec((ROW_BLK, D_MODEL), lambda i: (i, 0)),
            pl.BlockSpec((ROW_BLK, 1), lambda i: (i, 0)),
            pl.BlockSpec((ROW_BLK, 1), lambda i: (i, 0)),
        ],
        out_specs=pl.BlockSpec((ROW_BLK, D_MODEL), lambda i: (i, 0)),
        out_shape=jax.ShapeDtypeStruct((SEQ, D_MODEL), jnp.float32),
    )(rows1, rows2, w1, w2)


def kernel(hidden_states, Wg, W1, b1, W2, b2):
    x = hidden_states.reshape(SEQ, D_MODEL)
    slot1, slot2, w1, w2, laux, counts = _routing(x, Wg)
    slot1 = slot1.reshape(SEQ)
    slot2 = slot2.reshape(SEQ)
    expert_in = _dispatch(x, slot1, slot2)
    expert_out = _ffn(expert_in, W1, b1, W2, b2)
    rows1, rows2 = _gather(expert_out, slot1, slot2)
    out = _combine(rows1, rows2, w1, w2)
    return out.reshape(hidden_states.shape), laux[0, 0], counts.reshape(E)


# speedup vs baseline: 1.2781x; 1.0012x over previous
import functools

import jax
import jax.numpy as jnp
from jax import lax
from jax.experimental import pallas as pl
from jax.experimental.pallas import tpu as pltpu
from jax.experimental.pallas import tpu_sc as plsc

D_MODEL = 2048
D_FF = 4096
E = 16
SEQ = 2048
CAP = 320
NSLOT = E * CAP
TRASH = NSLOT
NSLOT_PAD = NSLOT + 8

NC = 2
NS = 16
NW = NC * NS
TOK_PER_W = SEQ // NW
CHUNK = 16


def _routing_body(x_ref, wg_ref, slot1_ref, slot2_ref, w1_ref, w2_ref, laux_ref,
                  cnt_ref):
    x = x_ref[...]
    wg = wg_ref[...]
    logits = jnp.dot(x, wg, preferred_element_type=jnp.float32)

    m = jnp.max(logits, axis=1, keepdims=True)
    eg = jnp.exp(logits - m)
    gates = eg / jnp.sum(eg, axis=1, keepdims=True)

    lane = lax.broadcasted_iota(jnp.int32, (SEQ, E), 1)
    idx1 = jnp.min(jnp.where(logits == m, lane, E), axis=1, keepdims=True)
    mask1 = (lane == idx1).astype(jnp.float32)
    logits2 = jnp.where(mask1 > 0, -jnp.inf, logits)
    m2 = jnp.max(logits2, axis=1, keepdims=True)
    idx2 = jnp.min(jnp.where(logits2 == m2, lane, E), axis=1, keepdims=True)
    mask2 = (lane == idx2).astype(jnp.float32)

    row = lax.broadcasted_iota(jnp.int32, (SEQ, SEQ), 0)
    col = lax.broadcasted_iota(jnp.int32, (SEQ, SEQ), 1)
    tri = (col <= row).astype(jnp.float32)
    cs1 = jnp.dot(tri, mask1, preferred_element_type=jnp.float32)
    cs2 = jnp.dot(tri, mask2, preferred_element_type=jnp.float32)
    n1 = jnp.sum(mask1, axis=0, keepdims=True)
    loc1 = cs1 - 1.0
    loc2 = cs2 - 1.0 + n1

    me = jnp.mean(gates, axis=0, keepdims=True)
    ce = jnp.mean(mask1, axis=0, keepdims=True)
    laux_ref[...] = jnp.sum(me * ce, axis=1, keepdims=True) * float(E * E)

    mask1d = mask1 * (loc1 < CAP).astype(jnp.float32)
    mask2d = mask2 * (loc2 < CAP).astype(jnp.float32)
    pos1 = jnp.sum(loc1 * mask1d, axis=1, keepdims=True).astype(jnp.int32)
    pos2 = jnp.sum(loc2 * mask2d, axis=1, keepdims=True).astype(jnp.int32)
    keep1 = jnp.sum(mask1d, axis=1, keepdims=True)
    keep2 = jnp.sum(mask2d, axis=1, keepdims=True)

    g1 = jnp.sum(gates * mask1d, axis=1, keepdims=True)
    g2 = jnp.sum(gates * mask2d, axis=1, keepdims=True)
    denom = g1 + g2
    denom = jnp.where(denom < 1e-9, 1.0, denom)
    w1_ref[...] = g1 / denom
    w2_ref[...] = g2 / denom

    cnt_ref[...] = jnp.sum(mask1d + mask2d, axis=0, keepdims=True).astype(jnp.int32)

    slot1_ref[...] = jnp.where(keep1 > 0, idx1 * CAP + pos1, TRASH)
    slot2_ref[...] = jnp.where(keep2 > 0, idx2 * CAP + pos2, TRASH)


def _routing(x, wg):
    return pl.pallas_call(
        _routing_body,
        out_shape=(
            jax.ShapeDtypeStruct((SEQ, 1), jnp.int32),
            jax.ShapeDtypeStruct((SEQ, 1), jnp.int32),
            jax.ShapeDtypeStruct((SEQ, 1), jnp.float32),
            jax.ShapeDtypeStruct((SEQ, 1), jnp.float32),
            jax.ShapeDtypeStruct((1, 1), jnp.float32),
            jax.ShapeDtypeStruct((1, E), jnp.int32),
        ),
    )(x, wg)


def _dispatch_body(x_hbm, slot1_hbm, slot2_hbm, eout_hbm, buf, idx1_v, idx2_v, sem):
    wid = lax.axis_index("s") * NC + lax.axis_index("c")
    base = wid * TOK_PER_W
    for ch in range(TOK_PER_W // CHUNK):
        off = base + ch * CHUNK
        pltpu.sync_copy(x_hbm.at[pl.ds(off, CHUNK)], buf)
        pltpu.sync_copy(slot1_hbm.at[pl.ds(off, CHUNK)], idx1_v)
        pltpu.sync_copy(slot2_hbm.at[pl.ds(off, CHUNK)], idx2_v)
        c1 = pltpu.async_copy(buf, eout_hbm.at[idx1_v], sem)
        c2 = pltpu.async_copy(buf, eout_hbm.at[idx2_v], sem)
        c1.wait()
        c2.wait()


def _dispatch(x, slot1, slot2):
    mesh = plsc.VectorSubcoreMesh(core_axis_name="c", subcore_axis_name="s")
    return pl.kernel(
        _dispatch_body,
        out_type=jax.ShapeDtypeStruct((NSLOT_PAD, D_MODEL), jnp.float32),
        mesh=mesh,
        scratch_types=[
            pltpu.VMEM((CHUNK, D_MODEL), jnp.float32),
            pltpu.VMEM((CHUNK,), jnp.int32),
            pltpu.VMEM((CHUNK,), jnp.int32),
            pltpu.SemaphoreType.DMA,
        ],
    )(x, slot1, slot2)


F_BLK = 1024
NF = D_FF // F_BLK


def _ffn_body(a_ref, w1_ref, b1_ref, w2_ref, b2_ref, out_ref):
    f = pl.program_id(1)
    h = jnp.dot(a_ref[...], w1_ref[0], preferred_element_type=jnp.float32,
                precision=lax.Precision.DEFAULT)
    h = jnp.maximum(h + b1_ref[0], 0.0)
    contrib = jnp.dot(h, w2_ref[0], preferred_element_type=jnp.float32,
                      precision=lax.Precision.DEFAULT)

    @pl.when(f == 0)
    def _():
        out_ref[...] = contrib + b2_ref[0]

    @pl.when(f != 0)
    def _():
        out_ref[...] += contrib


def _ffn(expert_in, w1, b1, w2, b2):
    return pl.pallas_call(
        _ffn_body,
        grid=(E, NF),
        in_specs=[
            pl.BlockSpec((CAP, D_MODEL), lambda e, f: (e, 0)),
            pl.BlockSpec((1, D_MODEL, F_BLK), lambda e, f: (e, 0, f)),
            pl.BlockSpec((1, 1, F_BLK), lambda e, f: (e, 0, f)),
            pl.BlockSpec((1, F_BLK, D_MODEL), lambda e, f: (e, f, 0)),
            pl.BlockSpec((1, 1, D_MODEL), lambda e, f: (e, 0, 0)),
        ],
        out_specs=pl.BlockSpec((CAP, D_MODEL), lambda e, f: (e, 0)),
        out_shape=jax.ShapeDtypeStruct((NSLOT_PAD, D_MODEL), jnp.float32),
        compiler_params=pltpu.CompilerParams(
            dimension_semantics=("arbitrary", "arbitrary"),
        ),
    )(expert_in, w1, b1.reshape(E, 1, D_FF), w2, b2.reshape(E, 1, D_MODEL))


def _gather_body(eo_hbm, slot1_hbm, slot2_hbm, g1_hbm, g2_hbm, buf1, buf2,
                 idx1_v, idx2_v, sem):
    wid = lax.axis_index("s") * NC + lax.axis_index("c")
    base = wid * TOK_PER_W
    for ch in range(TOK_PER_W // CHUNK):
        off = base + ch * CHUNK
        pltpu.sync_copy(slot1_hbm.at[pl.ds(off, CHUNK)], idx1_v)
        pltpu.sync_copy(slot2_hbm.at[pl.ds(off, CHUNK)], idx2_v)
        c1 = pltpu.async_copy(eo_hbm.at[idx1_v], buf1, sem)
        c2 = pltpu.async_copy(eo_hbm.at[idx2_v], buf2, sem)
        c1.wait()
        c2.wait()
        pltpu.sync_copy(buf1, g1_hbm.at[pl.ds(off, CHUNK)])
        pltpu.sync_copy(buf2, g2_hbm.at[pl.ds(off, CHUNK)])


def _gather(expert_out, slot1, slot2):
    mesh = plsc.VectorSubcoreMesh(core_axis_name="c", subcore_axis_name="s")
    return pl.kernel(
        _gather_body,
        out_type=(
            jax.ShapeDtypeStruct((SEQ, D_MODEL), jnp.float32),
            jax.ShapeDtypeStruct((SEQ, D_MODEL), jnp.float32),
        ),
        mesh=mesh,
        scratch_types=[
            pltpu.VMEM((CHUNK, D_MODEL), jnp.float32),
            pltpu.VMEM((CHUNK, D_MODEL), jnp.float32),
            pltpu.VMEM((CHUNK,), jnp.int32),
            pltpu.VMEM((CHUNK,), jnp.int32),
            pltpu.SemaphoreType.DMA,
        ],
    )(expert_out, slot1, slot2)


ROW_BLK = 256


def _combine_body(r1_ref, r2_ref, w1_ref, w2_ref, out_ref):
    w1 = w1_ref[...]
    w2 = w2_ref[...]
    t1 = jnp.where(w1 == 0.0, 0.0, w1 * r1_ref[...])
    t2 = jnp.where(w2 == 0.0, 0.0, w2 * r2_ref[...])
    out_ref[...] = t1 + t2


def _combine(rows1, rows2, w1, w2):
    return pl.pallas_call(
        _combine_body,
        grid=(SEQ // ROW_BLK,),
        in_specs=[
            pl.BlockSpec((ROW_BLK, D_MODEL), lambda i: (i, 0)),
            pl.BlockSpec((ROW_BLK, D_MODEL), lambda i: (i, 0)),
            pl.BlockSpec((ROW_BLK, 1), lambda i: (i, 0)),
            pl.BlockSpec((ROW_BLK, 1), lambda i: (i, 0)),
        ],
        out_specs=pl.BlockSpec((ROW_BLK, D_MODEL), lambda i: (i, 0)),
        out_shape=jax.ShapeDtypeStruct((SEQ, D_MODEL), jnp.float32),
    )(rows1, rows2, w1, w2)


def kernel(hidden_states, Wg, W1, b1, W2, b2):
    x = hidden_states.reshape(SEQ, D_MODEL)
    slot1, slot2, w1, w2, laux, counts = _routing(x, Wg)
    slot1 = slot1.reshape(SEQ)
    slot2 = slot2.reshape(SEQ)
    expert_in = _dispatch(x, slot1, slot2)
    expert_out = _ffn(expert_in, W1, b1, W2, b2)
    rows1, rows2 = _gather(expert_out, slot1, slot2)
    out = _combine(rows1, rows2, w1, w2)
    return out.reshape(hidden_states.shape), laux[0, 0], counts.reshape(E)
